# baseline (speedup 1.0000x reference)
import functools

import jax
import jax.numpy as jnp
from jax import lax
from jax.experimental import pallas as pl
from jax.experimental.pallas import tpu as pltpu
from jax.experimental.pallas import tpu_sc as plsc

N_NODES = 10000
N_PAD = 10240
N_CHUNKS = 2500
CHUNK = 128
MAX_CHUNKS = 79
ROWS_PER_TILE = N_PAD // 16

_mesh = plsc.VectorSubcoreMesh(core_axis_name="c", subcore_axis_name="s")
_sc_params = pltpu.CompilerParams(use_tc_tiling_on_sc=False)


def _tile_chunks(w):
    n = jnp.where(w < 4, 79, 78)
    off = 78 * w + jnp.minimum(w, 4)
    lo = jnp.where(w < 4, 0, 1)
    return n, off - lo, lo


@functools.partial(
    pl.kernel,
    mesh=_mesh,
    out_type=jax.ShapeDtypeStruct((2, N_PAD), jnp.float32),
    scratch_types=[
        pltpu.VMEM((MAX_CHUNKS, CHUNK), jnp.int32),
        pltpu.VMEM((CHUNK,), jnp.float32),
        pltpu.VMEM((ROWS_PER_TILE,), jnp.float32),
        pltpu.VMEM_SHARED((N_PAD,), jnp.float32),
    ],
    compiler_params=_sc_params,
)
def _deg_kernel(dst_hbm, part_hbm, idxd, ones_v, stage, acc):
    c = lax.axis_index("c")
    s = lax.axis_index("s")
    w = c * 16 + s
    n, win, lo = _tile_chunks(w)
    pltpu.sync_copy(dst_hbm.at[pl.ds(win, MAX_CHUNKS)], idxd)

    def fill_ones(j, carry):
        ones_v[pl.ds(j * 16, 16)] = jnp.ones((16,), jnp.float32)
        return carry

    lax.fori_loop(0, CHUNK // 16, fill_ones, 0)

    def zero_row(j, carry):
        stage[pl.ds(j * 16, 16)] = jnp.zeros((16,), jnp.float32)
        return carry

    lax.fori_loop(0, ROWS_PER_TILE // 16, zero_row, 0)
    pltpu.sync_copy(stage, acc.at[pl.ds(s * ROWS_PER_TILE, ROWS_PER_TILE)])
    plsc.subcore_barrier()

    def body(ch, carry):
        pltpu.sync_copy(ones_v, acc.at[idxd.at[ch]], add=True)
        return carry

    lax.fori_loop(lo, lo + n, body, 0)
    plsc.subcore_barrier()
    pltpu.sync_copy(acc.at[pl.ds(s * ROWS_PER_TILE, ROWS_PER_TILE)], stage)
    pltpu.sync_copy(stage, part_hbm.at[c, pl.ds(s * ROWS_PER_TILE, ROWS_PER_TILE)])


@functools.partial(
    pl.kernel,
    mesh=_mesh,
    out_type=jax.ShapeDtypeStruct((2, N_PAD, 16), jnp.float32),
    scratch_types=[
        pltpu.VMEM((MAX_CHUNKS, CHUNK), jnp.int32),
        pltpu.VMEM((MAX_CHUNKS, CHUNK), jnp.int32),
        pltpu.VMEM((8, CHUNK, 16), jnp.float32),
        pltpu.VMEM((ROWS_PER_TILE, 16), jnp.float32),
        pltpu.VMEM_SHARED((N_PAD, 16), jnp.float32),
        pltpu.SemaphoreType.DMA,
        pltpu.SemaphoreType.DMA,
    ],
    compiler_params=_sc_params,
)
def _agg_kernel(src_hbm, dst_hbm, g_hbm, part_hbm, idxs, idxd, rows, stage, acc,
                sem_g, sem_s):
    c = lax.axis_index("c")
    s = lax.axis_index("s")
    w = c * 16 + s
    n, win, lo = _tile_chunks(w)
    pltpu.sync_copy(src_hbm.at[pl.ds(win, MAX_CHUNKS)], idxs)
    pltpu.sync_copy(dst_hbm.at[pl.ds(win, MAX_CHUNKS)], idxd)

    def zero_row(j, carry):
        stage[j, :] = jnp.zeros((16,), jnp.float32)
        return carry

    lax.fori_loop(0, ROWS_PER_TILE, zero_row, 0)
    pltpu.sync_copy(stage, acc.at[pl.ds(s * ROWS_PER_TILE, ROWS_PER_TILE)])
    plsc.subcore_barrier()

    for b in range(4):
        ch0 = lo + b
        pltpu.async_copy(g_hbm.at[idxs.at[ch0]], rows.at[ch0 % 8], sem_g)

    def body(ch, carry):
        slot = ch % 8
        pltpu.make_async_copy(g_hbm.at[idxs.at[ch]], rows.at[slot], sem_g).wait()
        pltpu.async_copy(rows.at[slot], acc.at[idxd.at[ch]], sem_s, add=True)

        @pl.when(ch + 4 < lo + n)
        def _():
            nslot = (ch + 4) % 8

            @pl.when(ch >= lo + 4)
            def _():
                pltpu.make_async_copy(rows.at[nslot], acc.at[idxd.at[ch - 4]],
                                      sem_s).wait()

            pltpu.async_copy(g_hbm.at[idxs.at[ch + 4]], rows.at[nslot], sem_g)

        return carry

    lax.fori_loop(lo, lo + n, body, 0)

    def drain(i, carry):
        pltpu.make_async_copy(rows.at[i], acc.at[idxd.at[lo]], sem_s).wait()
        return carry

    lax.fori_loop(0, 8, drain, 0)
    plsc.subcore_barrier()
    pltpu.sync_copy(acc.at[pl.ds(s * ROWS_PER_TILE, ROWS_PER_TILE)], stage)
    pltpu.sync_copy(stage, part_hbm.at[c, pl.ds(s * ROWS_PER_TILE, ROWS_PER_TILE)])


def _tc1(x_ref, w_ref, degp_ref, g_ref, dis_ref):
    deg = degp_ref[0] + degp_ref[1] + 1.0
    dis = lax.rsqrt(deg)
    h = jnp.dot(x_ref[...], w_ref[...], preferred_element_type=jnp.float32)
    g_ref[:N_NODES, :] = h * dis[:N_NODES]
    g_ref[N_NODES:, :] = jnp.zeros((N_PAD - N_NODES, 16), jnp.float32)
    dis_ref[...] = dis


def _tc2(p_ref, g1_ref, dis_ref, w2_ref, b1_ref, g2_ref):
    z = dis_ref[...] * (p_ref[0] + p_ref[1] + g1_ref[...]) + b1_ref[...]
    a = jnp.maximum(z, 0.0)
    h2 = jnp.dot(a, w2_ref[...], preferred_element_type=jnp.float32)
    g2_ref[...] = h2 * dis_ref[...]


def _tc3(p_ref, g2_ref, dis_ref, b2_ref, o_ref):
    z = (dis_ref[:N_NODES] * (p_ref[0, :N_NODES] + p_ref[1, :N_NODES]
                              + g2_ref[:N_NODES]) + b2_ref[...])
    o_ref[...] = jax.nn.sigmoid(z)


def kernel(x, edge_index, W1, b1, W2, b2):
    f32 = jnp.float32
    ei = edge_index.astype(jnp.int32)
    src2d = ei[0].reshape(N_CHUNKS, CHUNK)
    dst2d = ei[1].reshape(N_CHUNKS, CHUNK)

    degp = _deg_kernel(dst2d).reshape(2, N_PAD, 1)

    g1, dis = pl.pallas_call(
        _tc1,
        out_shape=[
            jax.ShapeDtypeStruct((N_PAD, 16), f32),
            jax.ShapeDtypeStruct((N_PAD, 1), f32),
        ],
    )(x, W1, degp)

    p1 = _agg_kernel(src2d, dst2d, g1)

    g2 = pl.pallas_call(
        _tc2,
        out_shape=jax.ShapeDtypeStruct((N_PAD, 16), f32),
    )(p1, g1, dis, W2, b1.reshape(1, 16))

    p2 = _agg_kernel(src2d, dst2d, g2)

    return pl.pallas_call(
        _tc3,
        out_shape=jax.ShapeDtypeStruct((N_NODES, 16), f32),
    )(p2, g2, dis, b2.reshape(1, 16))


# speedup vs baseline: 1.0081x; 1.0081x over previous
import functools

import jax
import jax.numpy as jnp
from jax import lax
from jax.experimental import pallas as pl
from jax.experimental.pallas import tpu as pltpu
from jax.experimental.pallas import tpu_sc as plsc

N_NODES = 10000
N_PAD = 10240
N_CHUNKS = 2500
CHUNK = 128
MAX_CHUNKS = 79
ROWS_PER_TILE = N_PAD // 16

_mesh = plsc.VectorSubcoreMesh(core_axis_name="c", subcore_axis_name="s")
_sc_params = pltpu.CompilerParams(use_tc_tiling_on_sc=False,
                                  needs_layout_passes=False)


def _tile_chunks(w):
    n = jnp.where(w < 4, 79, 78)
    off = 78 * w + jnp.minimum(w, 4)
    lo = jnp.where(w < 4, 0, 1)
    return n, off - lo, lo


@functools.partial(
    pl.kernel,
    mesh=_mesh,
    out_type=jax.ShapeDtypeStruct((2, N_PAD, 16), jnp.float32),
    scratch_types=[
        pltpu.VMEM((MAX_CHUNKS, CHUNK), jnp.int32),
        pltpu.VMEM((CHUNK,), jnp.float32),
        pltpu.VMEM((ROWS_PER_TILE,), jnp.float32),
        pltpu.VMEM((ROWS_PER_TILE, 16), jnp.float32),
        pltpu.VMEM_SHARED((N_PAD,), jnp.float32),
    ],
    compiler_params=_sc_params,
)
def _deg_kernel(dst_hbm, part_hbm, idxd, ones_v, stage, stage2d, acc):
    c = lax.axis_index("c")
    s = lax.axis_index("s")
    w = c * 16 + s
    n, win, lo = _tile_chunks(w)
    pltpu.sync_copy(dst_hbm.at[pl.ds(win, MAX_CHUNKS)], idxd)

    def fill_ones(j, carry):
        ones_v[pl.ds(j * 16, 16)] = jnp.ones((16,), jnp.float32)
        return carry

    lax.fori_loop(0, CHUNK // 16, fill_ones, 0)

    def zero_row(j, carry):
        stage[pl.ds(j * 16, 16)] = jnp.zeros((16,), jnp.float32)
        return carry

    lax.fori_loop(0, ROWS_PER_TILE // 16, zero_row, 0)
    pltpu.sync_copy(stage, acc.at[pl.ds(s * ROWS_PER_TILE, ROWS_PER_TILE)])
    plsc.subcore_barrier()

    def body(ch, carry):
        pltpu.sync_copy(ones_v, acc.at[idxd.at[ch]], add=True)
        return carry

    lax.fori_loop(lo, lo + n, body, 0)
    plsc.subcore_barrier()
    pltpu.sync_copy(acc.at[pl.ds(s * ROWS_PER_TILE, ROWS_PER_TILE)], stage)

    def splat(j, carry):
        stage2d[j, :] = plsc.load_gather(stage, [jnp.full((16,), j, jnp.int32)])
        return carry

    lax.fori_loop(0, ROWS_PER_TILE, splat, 0)
    pltpu.sync_copy(stage2d, part_hbm.at[c, pl.ds(s * ROWS_PER_TILE, ROWS_PER_TILE)])


@functools.partial(
    pl.kernel,
    mesh=_mesh,
    out_type=jax.ShapeDtypeStruct((2, N_PAD, 16), jnp.float32),
    scratch_types=[
        pltpu.VMEM((MAX_CHUNKS, CHUNK), jnp.int32),
        pltpu.VMEM((MAX_CHUNKS, CHUNK), jnp.int32),
        pltpu.VMEM((8, CHUNK, 16), jnp.float32),
        pltpu.VMEM((ROWS_PER_TILE, 16), jnp.float32),
        pltpu.VMEM_SHARED((N_PAD, 16), jnp.float32),
        pltpu.SemaphoreType.DMA,
        pltpu.SemaphoreType.DMA,
    ],
    compiler_params=_sc_params,
)
def _agg_kernel(src_hbm, dst_hbm, g_hbm, part_hbm, idxs, idxd, rows, stage, acc,
                sem_g, sem_s):
    c = lax.axis_index("c")
    s = lax.axis_index("s")
    w = c * 16 + s
    n, win, lo = _tile_chunks(w)
    pltpu.sync_copy(src_hbm.at[pl.ds(win, MAX_CHUNKS)], idxs)
    pltpu.sync_copy(dst_hbm.at[pl.ds(win, MAX_CHUNKS)], idxd)

    def zero_row(j, carry):
        stage[j, :] = jnp.zeros((16,), jnp.float32)
        return carry

    lax.fori_loop(0, ROWS_PER_TILE, zero_row, 0)
    pltpu.sync_copy(stage, acc.at[pl.ds(s * ROWS_PER_TILE, ROWS_PER_TILE)])
    plsc.subcore_barrier()

    for b in range(4):
        ch0 = lo + b
        pltpu.async_copy(g_hbm.at[idxs.at[ch0]], rows.at[ch0 % 8], sem_g)

    def body(ch, carry):
        slot = ch % 8
        pltpu.make_async_copy(g_hbm.at[idxs.at[ch]], rows.at[slot], sem_g).wait()
        pltpu.async_copy(rows.at[slot], acc.at[idxd.at[ch]], sem_s, add=True)

        @pl.when(ch + 4 < lo + n)
        def _():
            nslot = (ch + 4) % 8

            @pl.when(ch >= lo + 4)
            def _():
                pltpu.make_async_copy(rows.at[nslot], acc.at[idxd.at[ch - 4]],
                                      sem_s).wait()

            pltpu.async_copy(g_hbm.at[idxs.at[ch + 4]], rows.at[nslot], sem_g)

        return carry

    lax.fori_loop(lo, lo + n, body, 0)

    def drain(i, carry):
        pltpu.make_async_copy(rows.at[i], acc.at[idxd.at[lo]], sem_s).wait()
        return carry

    lax.fori_loop(0, 8, drain, 0)
    plsc.subcore_barrier()
    pltpu.sync_copy(acc.at[pl.ds(s * ROWS_PER_TILE, ROWS_PER_TILE)], stage)
    pltpu.sync_copy(stage, part_hbm.at[c, pl.ds(s * ROWS_PER_TILE, ROWS_PER_TILE)])


def _tc1(x_ref, w_ref, degp_ref, g_ref, dis_ref):
    deg = degp_ref[0] + degp_ref[1] + 1.0
    dis16 = lax.rsqrt(deg)
    h = jnp.dot(x_ref[...], w_ref[...], preferred_element_type=jnp.float32)
    g_ref[:N_NODES, :] = h * dis16[:N_NODES]
    g_ref[N_NODES:, :] = jnp.zeros((N_PAD - N_NODES, 16), jnp.float32)
    dis_ref[...] = dis16


def _tc2(p_ref, g1_ref, dis_ref, w2_ref, b1_ref, g2_ref):
    z = dis_ref[...] * (p_ref[0] + p_ref[1] + g1_ref[...]) + b1_ref[...]
    a = jnp.maximum(z, 0.0)
    h2 = jnp.dot(a, w2_ref[...], preferred_element_type=jnp.float32)
    g2_ref[...] = h2 * dis_ref[...]


def _tc3(p_ref, g2_ref, dis_ref, b2_ref, o_ref):
    z = (dis_ref[:N_NODES] * (p_ref[0, :N_NODES] + p_ref[1, :N_NODES]
                              + g2_ref[:N_NODES]) + b2_ref[...])
    o_ref[...] = jax.nn.sigmoid(z)


def kernel(x, edge_index, W1, b1, W2, b2):
    f32 = jnp.float32
    ei = edge_index.astype(jnp.int32)
    src2d = ei[0].reshape(N_CHUNKS, CHUNK)
    dst2d = ei[1].reshape(N_CHUNKS, CHUNK)

    degp = _deg_kernel(dst2d)

    g1, dis = pl.pallas_call(
        _tc1,
        out_shape=[
            jax.ShapeDtypeStruct((N_PAD, 16), f32),
            jax.ShapeDtypeStruct((N_PAD, 16), f32),
        ],
    )(x, W1, degp)

    p1 = _agg_kernel(src2d, dst2d, g1)

    g2 = pl.pallas_call(
        _tc2,
        out_shape=jax.ShapeDtypeStruct((N_PAD, 16), f32),
    )(p1, g1, dis, W2, b1.reshape(1, 16))

    p2 = _agg_kernel(src2d, dst2d, g2)

    return pl.pallas_call(
        _tc3,
        out_shape=jax.ShapeDtypeStruct((N_NODES, 16), f32),
    )(p2, g2, dis, b2.reshape(1, 16))


# speedup vs baseline: 1.0666x; 1.0579x over previous
import functools

import jax
import jax.numpy as jnp
from jax import lax
from jax.experimental import pallas as pl
from jax.experimental.pallas import tpu as pltpu
from jax.experimental.pallas import tpu_sc as plsc

N_NODES = 10000
N_PAD = 10240
N_CHUNKS = 2500
CHUNK = 128
MAX_CHUNKS = 79
ROWS_PER_TILE = N_PAD // 16

_mesh = plsc.VectorSubcoreMesh(core_axis_name="c", subcore_axis_name="s")
_sc_params = pltpu.CompilerParams(use_tc_tiling_on_sc=False,
                                  needs_layout_passes=False)


def _tile_chunks(w):
    n = jnp.where(w < 4, 79, 78)
    off = 78 * w + jnp.minimum(w, 4)
    lo = jnp.where(w < 4, 0, 1)
    return n, off - lo, lo


@functools.partial(
    pl.kernel,
    mesh=_mesh,
    out_type=jax.ShapeDtypeStruct((2, N_PAD, 16), jnp.float32),
    scratch_types=[
        pltpu.VMEM((MAX_CHUNKS, CHUNK), jnp.int32),
        pltpu.VMEM((CHUNK,), jnp.float32),
        pltpu.VMEM((ROWS_PER_TILE,), jnp.float32),
        pltpu.VMEM((ROWS_PER_TILE, 16), jnp.float32),
        pltpu.VMEM_SHARED((N_PAD,), jnp.float32),
    ],
    compiler_params=_sc_params,
)
def _deg_kernel(edges_hbm, part_hbm, idxd, ones_v, stage, stage2d, acc):
    c = lax.axis_index("c")
    s = lax.axis_index("s")
    w = c * 16 + s
    n, win, lo = _tile_chunks(w)
    pltpu.sync_copy(edges_hbm.at[1, pl.ds(win, MAX_CHUNKS)], idxd)

    def fill_ones(j, carry):
        ones_v[pl.ds(j * 16, 16)] = jnp.ones((16,), jnp.float32)
        return carry

    lax.fori_loop(0, CHUNK // 16, fill_ones, 0)

    def zero_row(j, carry):
        stage[pl.ds(j * 16, 16)] = jnp.zeros((16,), jnp.float32)
        return carry

    lax.fori_loop(0, ROWS_PER_TILE // 16, zero_row, 0)
    pltpu.sync_copy(stage, acc.at[pl.ds(s * ROWS_PER_TILE, ROWS_PER_TILE)])
    plsc.subcore_barrier()

    def body(ch, carry):
        pltpu.sync_copy(ones_v, acc.at[idxd.at[ch]], add=True)
        return carry

    lax.fori_loop(lo, lo + n, body, 0)
    plsc.subcore_barrier()
    pltpu.sync_copy(acc.at[pl.ds(s * ROWS_PER_TILE, ROWS_PER_TILE)], stage)

    def splat(j, carry):
        stage2d[j, :] = plsc.load_gather(stage, [jnp.full((16,), j, jnp.int32)])
        return carry

    lax.fori_loop(0, ROWS_PER_TILE, splat, 0)
    pltpu.sync_copy(stage2d, part_hbm.at[c, pl.ds(s * ROWS_PER_TILE, ROWS_PER_TILE)])


@functools.partial(
    pl.kernel,
    mesh=_mesh,
    out_type=jax.ShapeDtypeStruct((2, N_PAD, 16), jnp.float32),
    scratch_types=[
        pltpu.VMEM((MAX_CHUNKS, CHUNK), jnp.int32),
        pltpu.VMEM((MAX_CHUNKS, CHUNK), jnp.int32),
        pltpu.VMEM((8, CHUNK, 16), jnp.float32),
        pltpu.VMEM((ROWS_PER_TILE, 16), jnp.float32),
        pltpu.VMEM_SHARED((N_PAD, 16), jnp.float32),
        pltpu.SemaphoreType.DMA,
        pltpu.SemaphoreType.DMA,
    ],
    compiler_params=_sc_params,
)
def _agg_kernel(edges_hbm, g_hbm, part_hbm, idxs, idxd, rows, stage, acc,
                sem_g, sem_s):
    c = lax.axis_index("c")
    s = lax.axis_index("s")
    w = c * 16 + s
    n, win, lo = _tile_chunks(w)
    pltpu.sync_copy(edges_hbm.at[0, pl.ds(win, MAX_CHUNKS)], idxs)
    pltpu.sync_copy(edges_hbm.at[1, pl.ds(win, MAX_CHUNKS)], idxd)

    def zero_row(j, carry):
        stage[j, :] = jnp.zeros((16,), jnp.float32)
        return carry

    lax.fori_loop(0, ROWS_PER_TILE, zero_row, 0)
    pltpu.sync_copy(stage, acc.at[pl.ds(s * ROWS_PER_TILE, ROWS_PER_TILE)])
    plsc.subcore_barrier()

    for b in range(4):
        ch0 = lo + b
        pltpu.async_copy(g_hbm.at[idxs.at[ch0]], rows.at[ch0 % 8], sem_g)

    def body(ch, carry):
        slot = ch % 8
        pltpu.make_async_copy(g_hbm.at[idxs.at[ch]], rows.at[slot], sem_g).wait()
        pltpu.async_copy(rows.at[slot], acc.at[idxd.at[ch]], sem_s, add=True)

        @pl.when(ch + 4 < lo + n)
        def _():
            nslot = (ch + 4) % 8

            @pl.when(ch >= lo + 4)
            def _():
                pltpu.make_async_copy(rows.at[nslot], acc.at[idxd.at[ch - 4]],
                                      sem_s).wait()

            pltpu.async_copy(g_hbm.at[idxs.at[ch + 4]], rows.at[nslot], sem_g)

        return carry

    lax.fori_loop(lo, lo + n, body, 0)

    def drain(i, carry):
        pltpu.make_async_copy(rows.at[i], acc.at[idxd.at[lo]], sem_s).wait()
        return carry

    lax.fori_loop(0, 8, drain, 0)
    plsc.subcore_barrier()
    pltpu.sync_copy(acc.at[pl.ds(s * ROWS_PER_TILE, ROWS_PER_TILE)], stage)
    pltpu.sync_copy(stage, part_hbm.at[c, pl.ds(s * ROWS_PER_TILE, ROWS_PER_TILE)])


def _tc1(x_ref, w_ref, degp_ref, g_ref, dis_ref):
    deg = degp_ref[0] + degp_ref[1] + 1.0
    dis16 = lax.rsqrt(deg)
    h = jnp.dot(x_ref[...], w_ref[...], preferred_element_type=jnp.float32)
    g_ref[:N_NODES, :] = h * dis16[:N_NODES]
    g_ref[N_NODES:, :] = jnp.zeros((N_PAD - N_NODES, 16), jnp.float32)
    dis_ref[...] = dis16


def _tc2(p_ref, g1_ref, dis_ref, w2_ref, b1_ref, g2_ref):
    z = dis_ref[...] * (p_ref[0] + p_ref[1] + g1_ref[...]) + b1_ref[...]
    a = jnp.maximum(z, 0.0)
    h2 = jnp.dot(a, w2_ref[...], preferred_element_type=jnp.float32)
    g2_ref[...] = h2 * dis_ref[...]


def _tc3(p_ref, g2_ref, dis_ref, b2_ref, o_ref):
    z = (dis_ref[:N_NODES] * (p_ref[0, :N_NODES] + p_ref[1, :N_NODES]
                              + g2_ref[:N_NODES]) + b2_ref[...])
    o_ref[...] = jax.nn.sigmoid(z)


def kernel(x, edge_index, W1, b1, W2, b2):
    f32 = jnp.float32
    edges3d = edge_index.astype(jnp.int32).reshape(2, N_CHUNKS, CHUNK)

    degp = _deg_kernel(edges3d)

    g1, dis = pl.pallas_call(
        _tc1,
        out_shape=[
            jax.ShapeDtypeStruct((N_PAD, 16), f32),
            jax.ShapeDtypeStruct((N_PAD, 16), f32),
        ],
    )(x, W1, degp)

    p1 = _agg_kernel(edges3d, g1)

    g2 = pl.pallas_call(
        _tc2,
        out_shape=jax.ShapeDtypeStruct((N_PAD, 16), f32),
    )(p1, g1, dis, W2, b1.reshape(1, 16))

    p2 = _agg_kernel(edges3d, g2)

    return pl.pallas_call(
        _tc3,
        out_shape=jax.ShapeDtypeStruct((N_NODES, 16), f32),
    )(p2, g2, dis, b2.reshape(1, 16))


# speedup vs baseline: 1.4854x; 1.3927x over previous
import functools

import jax
import jax.numpy as jnp
from jax import lax
from jax.experimental import pallas as pl
from jax.experimental.pallas import tpu as pltpu
from jax.experimental.pallas import tpu_sc as plsc

N_NODES = 10000
N_PAD = 10240
N_CHUNKS = 2500
CHUNK = 128
MAX_CHUNKS = 79
ROWS_PER_TILE = N_PAD // 16

_mesh = plsc.VectorSubcoreMesh(core_axis_name="c", subcore_axis_name="s")
_sc_params = pltpu.CompilerParams(use_tc_tiling_on_sc=False,
                                  needs_layout_passes=False)


def _tile_chunks(w):
    n = jnp.where(w < 4, 79, 78)
    off = 78 * w + jnp.minimum(w, 4)
    lo = jnp.where(w < 4, 0, 1)
    return n, off - lo, lo


@functools.partial(
    pl.kernel,
    mesh=_mesh,
    out_type=jax.ShapeDtypeStruct((2, N_PAD, 16), jnp.float32),
    scratch_types=[
        pltpu.VMEM((MAX_CHUNKS, CHUNK), jnp.int32),
        pltpu.VMEM((CHUNK,), jnp.float32),
        pltpu.VMEM((ROWS_PER_TILE,), jnp.float32),
        pltpu.VMEM((ROWS_PER_TILE, 16), jnp.float32),
        pltpu.VMEM_SHARED((N_PAD,), jnp.float32),
    ],
    compiler_params=_sc_params,
)
def _deg_kernel(edges_hbm, part_hbm, idxd, ones_v, stage, stage2d, acc):
    c = lax.axis_index("c")
    s = lax.axis_index("s")
    w = c * 16 + s
    n, win, lo = _tile_chunks(w)
    pltpu.sync_copy(edges_hbm.at[1, pl.ds(win, MAX_CHUNKS)], idxd)

    def fill_ones(j, carry):
        ones_v[pl.ds(j * 16, 16)] = jnp.ones((16,), jnp.float32)
        return carry

    lax.fori_loop(0, CHUNK // 16, fill_ones, 0)

    def zero_row(j, carry):
        stage[pl.ds(j * 16, 16)] = jnp.zeros((16,), jnp.float32)
        return carry

    lax.fori_loop(0, ROWS_PER_TILE // 16, zero_row, 0)
    pltpu.sync_copy(stage, acc.at[pl.ds(s * ROWS_PER_TILE, ROWS_PER_TILE)])
    plsc.subcore_barrier()

    def body(ch, carry):
        pltpu.sync_copy(ones_v, acc.at[idxd.at[ch]], add=True)
        return carry

    lax.fori_loop(lo, lo + n, body, 0)
    plsc.subcore_barrier()
    pltpu.sync_copy(acc.at[pl.ds(s * ROWS_PER_TILE, ROWS_PER_TILE)], stage)

    def splat(j, carry):
        stage2d[j, :] = plsc.load_gather(stage, [jnp.full((16,), j, jnp.int32)])
        return carry

    lax.fori_loop(0, ROWS_PER_TILE, splat, 0)
    pltpu.sync_copy(stage2d, part_hbm.at[c, pl.ds(s * ROWS_PER_TILE, ROWS_PER_TILE)])


@functools.partial(
    pl.kernel,
    mesh=_mesh,
    out_type=jax.ShapeDtypeStruct((2, N_PAD, 16), jnp.float32),
    scratch_types=[
        pltpu.VMEM((MAX_CHUNKS, CHUNK), jnp.int32),
        pltpu.VMEM((MAX_CHUNKS, CHUNK), jnp.int32),
        pltpu.VMEM((8, CHUNK, 16), jnp.float32),
        pltpu.VMEM((ROWS_PER_TILE, 16), jnp.float32),
        pltpu.VMEM_SHARED((N_PAD, 16), jnp.float32),
        pltpu.SemaphoreType.DMA,
        pltpu.SemaphoreType.DMA,
    ],
    compiler_params=_sc_params,
)
def _agg_kernel(edges_hbm, g_hbm, part_hbm, idxs, idxd, rows, stage, acc,
                sem_g, sem_s):
    c = lax.axis_index("c")
    s = lax.axis_index("s")
    w = c * 16 + s
    n, win, lo = _tile_chunks(w)
    pltpu.sync_copy(edges_hbm.at[0, pl.ds(win, MAX_CHUNKS)], idxs)
    pltpu.sync_copy(edges_hbm.at[1, pl.ds(win, MAX_CHUNKS)], idxd)

    def zero_row(j, carry):
        stage[j, :] = jnp.zeros((16,), jnp.float32)
        return carry

    lax.fori_loop(0, ROWS_PER_TILE, zero_row, 0)
    pltpu.sync_copy(stage, acc.at[pl.ds(s * ROWS_PER_TILE, ROWS_PER_TILE)])
    plsc.subcore_barrier()

    for b in range(4):
        ch0 = lo + b
        pltpu.async_copy(g_hbm.at[idxs.at[ch0]], rows.at[ch0 % 8], sem_g)

    def body(ch, carry):
        slot = ch % 8
        pltpu.make_async_copy(g_hbm.at[idxs.at[ch]], rows.at[slot], sem_g).wait()
        pltpu.async_copy(rows.at[slot], acc.at[idxd.at[ch]], sem_s, add=True)

        @pl.when(ch + 4 < lo + n)
        def _():
            nslot = (ch + 4) % 8

            @pl.when(ch >= lo + 4)
            def _():
                pltpu.make_async_copy(rows.at[nslot], acc.at[idxd.at[ch - 4]],
                                      sem_s).wait()

            pltpu.async_copy(g_hbm.at[idxs.at[ch + 4]], rows.at[nslot], sem_g)

        return carry

    lax.fori_loop(lo, lo + n, body, 0)

    def drain(i, carry):
        pltpu.make_async_copy(rows.at[i], acc.at[idxd.at[lo]], sem_s).wait()
        return carry

    lax.fori_loop(0, 8, drain, 0)
    plsc.subcore_barrier()
    pltpu.sync_copy(acc.at[pl.ds(s * ROWS_PER_TILE, ROWS_PER_TILE)], stage)
    pltpu.sync_copy(stage, part_hbm.at[c, pl.ds(s * ROWS_PER_TILE, ROWS_PER_TILE)])


NR = N_PAD // 8
NRV = N_NODES // 8


def _tc1(x_ref, w_ref, degp_ref, g_ref, dis_ref):
    deg = degp_ref[0] + degp_ref[1] + 1.0
    dis = lax.rsqrt(deg)
    h = jnp.dot(x_ref[...], w_ref[...], preferred_element_type=jnp.float32)
    g_ref[:NRV, :] = h * dis[:NRV]
    g_ref[NRV:, :] = jnp.zeros((NR - NRV, 128), jnp.float32)
    dis_ref[...] = dis


def _tc2(p_ref, g1_ref, dis_ref, w2_ref, b1_ref, g2_ref):
    z = dis_ref[...] * (p_ref[0] + p_ref[1] + g1_ref[...]) + b1_ref[...]
    a = jnp.maximum(z, 0.0)
    h2 = jnp.dot(a, w2_ref[...], preferred_element_type=jnp.float32)
    g2_ref[...] = h2 * dis_ref[...]


def _tc3(p_ref, g2_ref, dis_ref, b2_ref, o_ref):
    z = (dis_ref[:NRV] * (p_ref[0, :NRV] + p_ref[1, :NRV]
                          + g2_ref[:NRV]) + b2_ref[...])
    o_ref[...] = jax.nn.sigmoid(z)


def kernel(x, edge_index, W1, b1, W2, b2):
    f32 = jnp.float32
    edges3d = edge_index.astype(jnp.int32).reshape(2, N_CHUNKS, CHUNK)
    x128 = x.reshape(NRV, 8 * 128)
    eye8 = jnp.eye(8, dtype=f32)
    w1big = jnp.kron(eye8, W1)
    w2big = jnp.kron(eye8, W2)
    b1t = jnp.tile(b1, 8).reshape(1, 128)
    b2t = jnp.tile(b2, 8).reshape(1, 128)

    degp = _deg_kernel(edges3d).reshape(2, NR, 128)

    g1p, dis = pl.pallas_call(
        _tc1,
        out_shape=[
            jax.ShapeDtypeStruct((NR, 128), f32),
            jax.ShapeDtypeStruct((NR, 128), f32),
        ],
    )(x128, w1big, degp)

    p1 = _agg_kernel(edges3d, g1p.reshape(N_PAD, 16)).reshape(2, NR, 128)

    g2p = pl.pallas_call(
        _tc2,
        out_shape=jax.ShapeDtypeStruct((NR, 128), f32),
    )(p1, g1p, dis, w2big, b1t)

    p2 = _agg_kernel(edges3d, g2p.reshape(N_PAD, 16)).reshape(2, NR, 128)

    out = pl.pallas_call(
        _tc3,
        out_shape=jax.ShapeDtypeStruct((NRV, 128), f32),
    )(p2, g2p, dis, b2t)
    return out.reshape(N_NODES, 16)
